# baseline (speedup 1.0000x reference)
import jax
import jax.numpy as jnp
from jax.experimental import pallas as pl
from jax.experimental.pallas import tpu as pltpu

K = 100000
DIM = 128
B = 16384

_BR = 16384
_VAL_BLOCKS = B // _BR
_GRID = (K + _BR - 1) // _BR


def _push_body(bank_ref, value_ref, out_ref):
    i = pl.program_id(0)

    @pl.when(i < _VAL_BLOCKS)
    def _():
        out_ref[...] = value_ref[...]

    @pl.when(i >= _VAL_BLOCKS)
    def _():
        out_ref[...] = bank_ref[...]


@jax.jit
def kernel(bank, value):
    return pl.pallas_call(
        _push_body,
        grid=(_GRID,),
        in_specs=[
            pl.BlockSpec((_BR, DIM), lambda i: (jnp.maximum(i, _VAL_BLOCKS), 0)),
            pl.BlockSpec((_BR, DIM), lambda i: (jnp.minimum(i, _VAL_BLOCKS - 1), 0)),
        ],
        out_specs=pl.BlockSpec((_BR, DIM), lambda i: (i, 0)),
        out_shape=jax.ShapeDtypeStruct((K, DIM), jnp.float32),
    )(bank, value)
